# Initial kernel scaffold; baseline (speedup 1.0000x reference)
import jax
import jax.numpy as jnp
from jax.experimental import pallas as pl


def kernel(x, edge_index, W, att_src, att_dst, gat_bias, Wt, bt, gamma, beta):
    raise NotImplementedError("write your pallas kernel here")


# speedup vs baseline: 9.0376x; 9.0376x over previous
import functools

import jax
import jax.numpy as jnp
from jax import lax
from jax.experimental import pallas as pl
from jax.experimental.pallas import tpu as pltpu
from jax.experimental.pallas import tpu_sc as plsc

N_NODES = 10000
N_EDGES = 320000
E_REAL = N_EDGES + N_NODES
IN_DIM = 128
OUT_DIM = 128
HEADS = 8
DA = 144
NEG_SLOPE = 0.2

NB = 1000
K2 = 64
KA = 512
N_TILES = 16
PER_TILE = 20992
E_PAD = N_TILES * PER_TILE
C2 = PER_TILE // K2
CA = PER_TILE // KA
NPAD = 10240
STRIPE = NPAD // N_TILES


def _proj_body(x_ref, w_ref, asrc_ref, adst_ref, xpa_ref, at_ref, bt_ref):
    xp = jnp.dot(x_ref[...], w_ref[...], preferred_element_type=jnp.float32)
    xph = xp.reshape(NB, HEADS, OUT_DIM)
    a_s = jnp.sum(xph * asrc_ref[...][None, :, :], axis=-1)
    a_d = jnp.sum(xph * adst_ref[...][None, :, :], axis=-1)
    at_ref[...] = a_s
    bt_ref[...] = a_d
    pad = (lax.broadcasted_iota(jnp.int32, (NB, DA - OUT_DIM), 1) == 0)
    padv = pad.astype(jnp.float32)
    for h in range(HEADS):
        xpa_ref[h, :, 0:OUT_DIM] = xph[:, h, :]
        xpa_ref[h, :, OUT_DIM:DA] = padv


def _proj(x, W, att_src, att_dst):
    grid = N_NODES // NB
    return pl.pallas_call(
        _proj_body,
        grid=(grid,),
        in_specs=[
            pl.BlockSpec((NB, IN_DIM), lambda i: (i, 0)),
            pl.BlockSpec((IN_DIM, HEADS * OUT_DIM), lambda i: (0, 0)),
            pl.BlockSpec((HEADS, OUT_DIM), lambda i: (0, 0)),
            pl.BlockSpec((HEADS, OUT_DIM), lambda i: (0, 0)),
        ],
        out_specs=[
            pl.BlockSpec((HEADS, NB, DA), lambda i: (0, i, 0)),
            pl.BlockSpec((NB, HEADS), lambda i: (i, 0)),
            pl.BlockSpec((NB, HEADS), lambda i: (i, 0)),
        ],
        out_shape=[
            jax.ShapeDtypeStruct((HEADS, N_NODES, DA), jnp.float32),
            jax.ShapeDtypeStruct((N_NODES, HEADS), jnp.float32),
            jax.ShapeDtypeStruct((N_NODES, HEADS), jnp.float32),
        ],
    )(x, W, att_src, att_dst)


def _sc_edge_softmax_num(asrc_t, adst_t, src16, dst16, chunk, g):
    dcl = jnp.minimum(dst16, N_NODES - 1)
    e = plsc.load_gather(asrc_t, [src16]) + plsc.load_gather(adst_t, [dcl])
    e = jnp.where(e >= 0.0, e, NEG_SLOPE * e)
    s = jnp.exp(e)
    pos = g * 16 + lax.iota(jnp.int32, 16)
    ridx = chunk * 63 + pos - 1
    ok = (pos > 0) & (ridx < E_REAL)
    return jnp.where(ok, s, 0.0)


def _agg_body(xpaf, asrctf, adsttf, srcp, dstp, uf,
              asrc_t, adst_t, src_v, dst_v, s_v, rows_v, zbuf, usp,
              sem):
    core = lax.axis_index("c")
    sid = lax.axis_index("s")
    z16 = jnp.zeros((16,), jnp.float32)
    for r in range(K2):
        for c in range(DA // 16):
            zbuf[r, pl.ds(c * 16, 16)] = z16

    def chunk_body(ci, hh):
        base = sid * PER_TILE + ci * K2
        chunk = sid * C2 + ci
        pltpu.sync_copy(srcp.at[pl.ds(base, K2)], src_v)
        pltpu.sync_copy(dstp.at[pl.ds(base, K2)], dst_v)
        hoff = hh * N_NODES
        descs = []
        for g in range(K2 // 16):
            sl = pl.ds(g * 16, 16)
            sv = src_v[sl]
            s = _sc_edge_softmax_num(asrc_t, adst_t, sv, dst_v[sl], chunk, g)
            s_v[sl] = s
            descs.append(pltpu.async_copy(xpaf.at[sv + hoff],
                                          rows_v.at[sl], sem))
        for dd in descs:
            dd.wait()
        for r in range(K2):
            sb = plsc.load_gather(s_v, [jnp.full((16,), r, jnp.int32)])
            for c in range(DA // 16):
                sl = pl.ds(c * 16, 16)
                rows_v[r, sl] = rows_v[r, sl] * sb
        pltpu.sync_copy(rows_v, usp.at[dst_v], add=True)
        return hh

    def slot_body(slot, _):
        hh = core * (HEADS // 2) + slot
        pltpu.sync_copy(asrctf.at[pl.ds(hh * N_NODES, N_NODES)], asrc_t)
        pltpu.sync_copy(adsttf.at[pl.ds(hh * N_NODES, N_NODES)], adst_t)
        soff = sid * STRIPE
        for k in range(STRIPE // K2):
            pltpu.sync_copy(zbuf, usp.at[pl.ds(soff + k * K2, K2)])
        plsc.subcore_barrier()
        lax.fori_loop(0, C2, chunk_body, hh)
        plsc.subcore_barrier()

        @pl.when(sid < N_TILES - 1)
        def _full():
            pltpu.sync_copy(usp.at[pl.ds(soff, STRIPE)],
                            uf.at[pl.ds(hh * N_NODES + soff, STRIPE)])

        @pl.when(sid == N_TILES - 1)
        def _tail():
            tail = N_NODES - (N_TILES - 1) * STRIPE
            pltpu.sync_copy(usp.at[pl.ds(soff, tail)],
                            uf.at[pl.ds(hh * N_NODES + soff, tail)])
        return 0

    lax.fori_loop(0, HEADS // 2, slot_body, 0)


def _aggregate(xpaf, asrctf, adsttf, srcp, dstp):
    mesh = plsc.VectorSubcoreMesh(core_axis_name="c", subcore_axis_name="s")
    f = pl.kernel(
        _agg_body,
        out_type=jax.ShapeDtypeStruct((HEADS * N_NODES, DA), jnp.float32),
        mesh=mesh,
        scratch_types=[
            pltpu.VMEM((N_NODES,), jnp.float32),
            pltpu.VMEM((N_NODES,), jnp.float32),
            pltpu.VMEM((K2,), jnp.int32),
            pltpu.VMEM((K2,), jnp.int32),
            pltpu.VMEM((K2,), jnp.float32),
            pltpu.VMEM((K2, DA), jnp.float32),
            pltpu.VMEM((K2, DA), jnp.float32),
            pltpu.VMEM_SHARED((NPAD, DA), jnp.float32),
            pltpu.SemaphoreType.DMA,
        ],
        compiler_params=pltpu.CompilerParams(needs_layout_passes=False,
                                             use_tc_tiling_on_sc=False),
    )
    return f(xpaf, asrctf, adsttf, srcp, dstp)


def _final_body(u_ref, x_ref, wt_ref, bt_ref, gb_ref, gm_ref, be_ref,
                h_ref, den_ref):
    u = u_ref[...]
    den = jnp.sum(u[:, :, OUT_DIM:DA], axis=-1) + 1e-16
    outh = u[:, :, 0:OUT_DIM] / den[:, :, None]
    hg = jnp.mean(outh, axis=0) + gb_ref[...]
    hv = hg + jnp.dot(x_ref[...], wt_ref[...],
                      preferred_element_type=jnp.float32) + bt_ref[...]
    mu = jnp.mean(hv, axis=-1, keepdims=True)
    var = jnp.mean((hv - mu) ** 2, axis=-1, keepdims=True)
    hn = (hv - mu) * lax.rsqrt(var + 1e-5) * gm_ref[...] + be_ref[...]
    h_ref[...] = jnp.maximum(hn, 0.0)
    den_ref[...] = den.T


def _final(u3, x, Wt, bt, gat_bias, gamma, beta):
    grid = N_NODES // NB
    return pl.pallas_call(
        _final_body,
        grid=(grid,),
        in_specs=[
            pl.BlockSpec((HEADS, NB, DA), lambda i: (0, i, 0)),
            pl.BlockSpec((NB, IN_DIM), lambda i: (i, 0)),
            pl.BlockSpec((IN_DIM, OUT_DIM), lambda i: (0, 0)),
            pl.BlockSpec((1, OUT_DIM), lambda i: (0, 0)),
            pl.BlockSpec((1, OUT_DIM), lambda i: (0, 0)),
            pl.BlockSpec((1, OUT_DIM), lambda i: (0, 0)),
            pl.BlockSpec((1, OUT_DIM), lambda i: (0, 0)),
        ],
        out_specs=[
            pl.BlockSpec((NB, OUT_DIM), lambda i: (i, 0)),
            pl.BlockSpec((NB, HEADS), lambda i: (i, 0)),
        ],
        out_shape=[
            jax.ShapeDtypeStruct((N_NODES, OUT_DIM), jnp.float32),
            jax.ShapeDtypeStruct((N_NODES, HEADS), jnp.float32),
        ],
    )(u3, x, Wt, bt, gat_bias, gamma, beta)


def _alpha_body(asrctf, adsttf, dentf, srcp, dstp, alphatf,
                asrc_t, adst_t, den_t, src_v, dst_v, a_v):
    core = lax.axis_index("c")
    sid = lax.axis_index("s")

    def chunk_body(ci, hh):
        base = sid * PER_TILE + ci * KA
        chunk0 = (sid * PER_TILE + ci * KA) // K2
        pltpu.sync_copy(srcp.at[pl.ds(base, KA)], src_v)
        pltpu.sync_copy(dstp.at[pl.ds(base, KA)], dst_v)
        for g in range(KA // 16):
            sl = pl.ds(g * 16, 16)
            dv = dst_v[sl]
            s = _sc_edge_softmax_num(asrc_t, adst_t, src_v[sl], dv,
                                     chunk0 + g // (K2 // 16), g % (K2 // 16))
            dcl = jnp.minimum(dv, N_NODES - 1)
            a_v[sl] = s / plsc.load_gather(den_t, [dcl])
        pltpu.sync_copy(a_v, alphatf.at[pl.ds(hh * E_PAD + base, KA)])
        return hh

    def slot_body(slot, _):
        hh = core * (HEADS // 2) + slot
        pltpu.sync_copy(asrctf.at[pl.ds(hh * N_NODES, N_NODES)], asrc_t)
        pltpu.sync_copy(adsttf.at[pl.ds(hh * N_NODES, N_NODES)], adst_t)
        pltpu.sync_copy(dentf.at[pl.ds(hh * N_NODES, N_NODES)], den_t)
        lax.fori_loop(0, CA, chunk_body, hh)
        return 0

    lax.fori_loop(0, HEADS // 2, slot_body, 0)


def _alpha(asrctf, adsttf, dentf, srcp, dstp):
    mesh = plsc.VectorSubcoreMesh(core_axis_name="c", subcore_axis_name="s")
    f = pl.kernel(
        _alpha_body,
        out_type=jax.ShapeDtypeStruct((HEADS * E_PAD,), jnp.float32),
        mesh=mesh,
        scratch_types=[
            pltpu.VMEM((N_NODES,), jnp.float32),
            pltpu.VMEM((N_NODES,), jnp.float32),
            pltpu.VMEM((N_NODES,), jnp.float32),
            pltpu.VMEM((KA,), jnp.int32),
            pltpu.VMEM((KA,), jnp.int32),
            pltpu.VMEM((KA,), jnp.float32),
        ],
        compiler_params=pltpu.CompilerParams(needs_layout_passes=False),
    )
    return f(asrctf, adsttf, dentf, srcp, dstp)


def _xpose_body(at_ref, out_ref):
    out_ref[...] = at_ref[...].T


def _alpha_xpose(alphat):
    eb = 2048
    grid = E_PAD // eb
    return pl.pallas_call(
        _xpose_body,
        grid=(grid,),
        in_specs=[pl.BlockSpec((HEADS, eb), lambda i: (0, i))],
        out_specs=pl.BlockSpec((eb, HEADS), lambda i: (i, 0)),
        out_shape=jax.ShapeDtypeStruct((E_PAD, HEADS), jnp.float32),
    )(alphat)


def kernel(x, edge_index, W, att_src, att_dst, gat_bias, Wt, bt, gamma, beta):
    n = x.shape[0]
    loop = jnp.arange(n, dtype=edge_index.dtype)
    src = jnp.concatenate([edge_index[0], loop])
    dst = jnp.concatenate([edge_index[1], loop])
    nch = E_PAD // K2
    cap = nch * 63
    zpad = jnp.zeros((cap - E_REAL,), src.dtype)
    src_r = jnp.concatenate([src, zpad]).reshape(nch, 63)
    dst_r = jnp.concatenate([dst, zpad]).reshape(nch, 63)
    srcp = jnp.concatenate(
        [jnp.zeros((nch, 1), src.dtype), src_r], axis=1).reshape(-1)
    dstp = jnp.concatenate(
        [jnp.full((nch, 1), NPAD - 1, dst.dtype), dst_r], axis=1).reshape(-1)

    xpa, asrc, adst = _proj(x, W, att_src, att_dst)
    xpaf = xpa.reshape(HEADS * N_NODES, DA)
    asrctf = asrc.T.reshape(HEADS * N_NODES)
    adsttf = adst.T.reshape(HEADS * N_NODES)

    uf = _aggregate(xpaf, asrctf, adsttf, srcp, dstp)
    u3 = uf.reshape(HEADS, N_NODES, DA)

    h, dent = _final(u3, x, Wt.reshape(IN_DIM, OUT_DIM), bt.reshape(1, OUT_DIM),
                     gat_bias.reshape(1, OUT_DIM), gamma.reshape(1, OUT_DIM),
                     beta.reshape(1, OUT_DIM))

    alphatf = _alpha(asrctf, adsttf, dent.T.reshape(HEADS * N_NODES), srcp, dstp)
    alphap = _alpha_xpose(alphatf.reshape(HEADS, E_PAD))
    alpha = alphap.reshape(nch, K2, HEADS)[:, 1:, :].reshape(cap, HEADS)[:E_REAL]

    edge_index_out = jnp.stack([src, dst])
    return h, edge_index_out, alpha
